# baseline (speedup 1.0000x reference)
import functools

import jax
import jax.numpy as jnp
from jax import lax
from jax.experimental import pallas as pl
from jax.experimental.pallas import tpu as pltpu
from jax.experimental.pallas import tpu_sc as plsc

_NC = 2
_NS = 16
_KP = 80


def _make_agg(N, D, E, with_deg, tc_tiling=True):
    NW = _NC * _NS
    epw = E // NW
    K = _KP
    assert epw % K == 0 and E % NW == 0
    nch = epw // K
    rpt = -(-N // _NS) // 8 * 8 or 8
    while N % rpt != 0 or rpt % 8 != 0:
        rpt += 8
    ncopy = N // rpt

    mesh = plsc.VectorSubcoreMesh(core_axis_name="c", subcore_axis_name="s")
    if with_deg:
        outs = (jax.ShapeDtypeStruct((_NC, N, D), jnp.float32),
                jax.ShapeDtypeStruct((_NC, N + 8), jnp.float32))
    else:
        outs = jax.ShapeDtypeStruct((_NC, N, D), jnp.float32)
    scratch = [
        pltpu.VMEM((K,), jnp.float32),
        pltpu.VMEM_SHARED((N + 8, D), jnp.float32),
        pltpu.VMEM_SHARED((N + 8,), jnp.float32),
        pltpu.VMEM((K, D), jnp.float32),
    ]
    scratch += [pltpu.VMEM((K,), jnp.int32) for _ in range(4)]
    scratch += [pltpu.SemaphoreType.DMA for _ in range(3)]

    def body(y_h, src_h, dst_h, znd_h, zdeg_h, *rest):
        if with_deg:
            z_out, deg_out = rest[0], rest[1]
            rest = rest[2:]
        else:
            z_out = rest[0]
            rest = rest[1:]
        (ones_v, z_sh, deg_sh, rows,
         sidx0, sidx1, didx0, didx1, gsem, isem0, isem1) = rest
        sidx = (sidx0, sidx1)
        didx = (didx0, didx1)
        isem = (isem0, isem1)
        c = lax.axis_index("c")
        s = lax.axis_index("s")
        wid = c * _NS + s
        base = wid * epw

        def idx_copies(j, q):
            off = pl.multiple_of(base + j * K, 8)
            return (pltpu.make_async_copy(src_h.at[pl.ds(off, K)],
                                          sidx[q], isem[q]),
                    pltpu.make_async_copy(dst_h.at[pl.ds(off, K)],
                                          didx[q], isem[q]))

        @pl.when(s < ncopy)
        def _():
            pltpu.sync_copy(znd_h.at[pl.ds(s * rpt, rpt)],
                            z_sh.at[pl.ds(s * rpt, rpt)])
        if with_deg:
            @pl.when(s == 0)
            def _():
                pltpu.sync_copy(zdeg_h, deg_sh)
            for j in range(K // 16):
                ones_v[pl.ds(16 * j, 16)] = jnp.full((16,), 1.0, jnp.float32)
        for cp in idx_copies(0, 0):
            cp.start()
        for cp in idx_copies(0, 0):
            cp.wait()
        plsc.subcore_barrier()

        def one_chunk(j, q, prefetch):
            if prefetch:
                nxt = idx_copies(j + 1, 1 - q)
                for cp in nxt:
                    cp.start()
            g = pltpu.make_async_copy(y_h.at[sidx[q]], rows, gsem)
            g.start()
            g.wait()
            pltpu.sync_copy(rows, z_sh.at[didx[q]], add=True)
            if with_deg:
                pltpu.sync_copy(ones_v, deg_sh.at[didx[q]], add=True)
            if prefetch:
                for cp in nxt:
                    cp.wait()

        def pair(i2, carry):
            one_chunk(2 * i2, 0, True)
            one_chunk(2 * i2 + 1, 1, True)
            return carry

        m = (nch - 1) // 2
        lax.fori_loop(0, m, pair, 0)
        if nch % 2 == 1:
            one_chunk(nch - 1, 0, False)
        else:
            one_chunk(nch - 2, 0, True)
            one_chunk(nch - 1, 1, False)
        plsc.subcore_barrier()

        @pl.when(s < ncopy)
        def _():
            pltpu.sync_copy(z_sh.at[pl.ds(s * rpt, rpt)],
                            z_out.at[c, pl.ds(s * rpt, rpt)])
        if with_deg:
            @pl.when(s == 0)
            def _():
                pltpu.sync_copy(deg_sh, deg_out.at[c])

    kwargs = {}
    if not tc_tiling:
        kwargs["compiler_params"] = pltpu.CompilerParams(
            use_tc_tiling_on_sc=False)
    return pl.kernel(body, mesh=mesh, out_type=outs, scratch_types=scratch,
                     **kwargs)


def _mm(x, w):
    def body(x_ref, w_ref, o_ref):
        o_ref[...] = jnp.dot(x_ref[...], w_ref[...],
                             preferred_element_type=jnp.float32)
    return pl.pallas_call(
        body,
        out_shape=jax.ShapeDtypeStruct((x.shape[0], w.shape[1]), jnp.float32),
    )(x, w)


def _combine(x, z2, deg2, Ws, b, g, be, rm, rv, Wn_next=None):
    Nn = x.shape[0]
    Dh = Ws.shape[1]

    def body(x_ref, z_ref, d_ref, ws_ref, b_ref, g_ref, be_ref, rm_ref,
             rv_ref, *rest):
        if Wn_next is not None:
            wn_ref, xo_ref, yo_ref = rest
        else:
            (xo_ref,) = rest
        z = z_ref[0] + z_ref[1]
        rs = 1.0 / jnp.maximum(d_ref[0] + d_ref[1], 1.0)
        h = (jnp.dot(x_ref[...], ws_ref[...],
                     preferred_element_type=jnp.float32)
             + b_ref[...] + z * rs)
        h = (h - rm_ref[...]) * (g_ref[...] * lax.rsqrt(rv_ref[...] + 1e-5)) \
            + be_ref[...]
        xo = jnp.maximum(h, 0.0)
        xo_ref[...] = xo
        if Wn_next is not None:
            yo_ref[...] = jnp.dot(xo, wn_ref[...],
                                  preferred_element_type=jnp.float32)

    if Wn_next is not None:
        Dn = Wn_next.shape[1]
        return pl.pallas_call(
            body,
            out_shape=(jax.ShapeDtypeStruct((Nn, Dh), jnp.float32),
                       jax.ShapeDtypeStruct((Nn, Dn), jnp.float32)),
        )(x, z2, deg2, Ws, b, g, be, rm, rv, Wn_next)
    return pl.pallas_call(
        body,
        out_shape=jax.ShapeDtypeStruct((Nn, Dh), jnp.float32),
    )(x, z2, deg2, Ws, b, g, be, rm, rv)


def _final(x, z2, deg2, Ws, b):
    Nn = x.shape[0]
    Do = Ws.shape[1]

    def body(x_ref, z_ref, d_ref, ws_ref, b_ref, o_ref):
        z = z_ref[0] + z_ref[1]
        rs = 1.0 / jnp.maximum(d_ref[0] + d_ref[1], 1.0)
        lg = (jnp.dot(x_ref[...], ws_ref[...],
                      preferred_element_type=jnp.float32)
              + z * rs + b_ref[...])
        m = jnp.max(lg, axis=-1, keepdims=True)
        lse = jnp.log(jnp.sum(jnp.exp(lg - m), axis=-1, keepdims=True)) + m
        o_ref[...] = lg - lse

    return pl.pallas_call(
        body,
        out_shape=jax.ShapeDtypeStruct((Nn, Do), jnp.float32),
    )(x, z2, deg2, Ws, b)


def kernel(graph, inputs, Ws0, Wn0, b0, Ws1, Wn1, b1, Ws2, Wn2, b2,
           g0, be0, rm0, rv0, g1, be1, rm1, rv1):
    src = graph[0]
    dst = graph[1]
    N, Din = inputs.shape
    E = src.shape[0]
    NW = _NC * _NS
    epw_r = E // NW
    ch = max(d for d in range(1, _KP + 1) if epw_r % d == 0)
    nch = epw_r // ch
    src = jnp.pad(src.reshape(NW, nch, ch),
                  ((0, 0), (0, 0), (0, _KP - ch))).reshape(-1)
    dst = jnp.pad(dst.reshape(NW, nch, ch),
                  ((0, 0), (0, 0), (0, _KP - ch)),
                  constant_values=N).reshape(-1)
    E = src.shape[0]
    Dh = Ws0.shape[1]
    Do = Ws2.shape[1]

    zeros_h = jnp.zeros((N, Dh), jnp.float32)
    zeros_o = jnp.zeros((N, Do), jnp.float32)
    zeros_d = jnp.zeros((N + 8,), jnp.float32)

    agg_deg = _make_agg(N, Dh, E, with_deg=True, tc_tiling=False)
    agg_h = _make_agg(N, Dh, E, with_deg=False, tc_tiling=False)
    agg_o = _make_agg(N, Do, E, with_deg=False, tc_tiling=False)

    y0 = _mm(inputs, Wn0)
    z0, deg2 = agg_deg(y0, src, dst, zeros_h, zeros_d)
    deg2 = deg2[:, :N].reshape(_NC, N, 1)
    x1, y1 = _combine(inputs, z0, deg2, Ws0, b0.reshape(1, -1),
                      g0.reshape(1, -1), be0.reshape(1, -1),
                      rm0.reshape(1, -1), rv0.reshape(1, -1), Wn1)
    z1 = agg_h(y1, src, dst, zeros_h, zeros_d)
    x2, y2 = _combine(x1, z1, deg2, Ws1, b1.reshape(1, -1),
                      g1.reshape(1, -1), be1.reshape(1, -1),
                      rm1.reshape(1, -1), rv1.reshape(1, -1), Wn2)
    z2 = agg_o(y2, src, dst, zeros_o, zeros_d)
    out = _final(x2, z2, deg2, Ws2, b2.reshape(1, -1))
    return (out, inputs, x2)
